# baseline (speedup 1.0000x reference)
import jax
import jax.numpy as jnp
from jax import lax
from jax.experimental import pallas as pl
from jax.experimental.pallas import tpu as pltpu
from jax.experimental.pallas import tpu_sc as plsc

N = 100000
E = 1600000
F_IN = 128
D = 32
G = 256

NC = 2
NS = 16
NW = NC * NS

CH = 512
SUB = CH // 128
PR = 2 * CH
EPAD = 1605632
EPS = EPAD // NS
NITER = EPS // PR
EPW = EPAD // NW
DCH = 7168
NPD = 100352

HALF = 16
NB = 2048
QR = NB // 4
PKN = NPD * D // 128
BKB = NB // 128


def _mesh():
  return plsc.VectorSubcoreMesh(core_axis_name="c", subcore_axis_name="s",
                                num_cores=NC, num_subcores=NS)


def _deg_body(dst_hbm, w_hbm, zero_hbm, out_hbm, acc, dstb, wb, sem):
  c = lax.axis_index("c")
  s = lax.axis_index("s")
  wid = s * NC + c
  pltpu.sync_copy(zero_hbm, acc)
  base = wid * EPW

  def chunk(k, _):
    off = base + k * DCH
    pltpu.async_copy(dst_hbm.at[pl.ds(off, DCH)], dstb, sem).wait()
    pltpu.async_copy(w_hbm.at[pl.ds(off, DCH)], wb, sem).wait()

    def inner(i, _):
      idx = dstb[pl.ds(i * 16, 16)]
      val = wb[pl.ds(i * 16, 16)]
      plsc.addupdate_scatter(acc, [idx], val)
      return 0

    lax.fori_loop(0, DCH // 16, inner, 0, unroll=8)
    return 0

  lax.fori_loop(0, EPW // DCH, chunk, 0)
  pltpu.sync_copy(acc, out_hbm.at[wid])


def _sc_degree(dstp, wp, zeros1):
  fn = pl.kernel(
      _deg_body,
      out_type=jax.ShapeDtypeStruct((NW, NPD), jnp.float32),
      mesh=_mesh(),
      scratch_types=[
          pltpu.VMEM((NPD,), jnp.float32),
          pltpu.VMEM((DCH,), jnp.int32),
          pltpu.VMEM((DCH,), jnp.float32),
          pltpu.SemaphoreType.DMA,
      ],
      compiler_params=pltpu.CompilerParams(needs_layout_passes=False),
  )
  return fn(dstp, wp, zeros1)


def _edge_body(hview, src2, dst2, w_hbm, zero_hbm, out,
               idxb, dstb, wb, rows, lsem, gsem, ssem, acc):
  c = lax.axis_index("c")
  s = lax.axis_index("s")

  for k in range(NPD // PR):
    @pl.when(k % NS == s)
    def _():
      pltpu.sync_copy(zero_hbm.at[pl.ds(k * PR, PR)], acc.at[pl.ds(k * PR, PR)])
  plsc.subcore_barrier()

  base_rows = s * (EPS // 128)

  def scale_half(h):
    @plsc.parallel_loop(h * (CH // 16), (h + 1) * (CH // 16), 1, unroll=4)
    def _(i):
      wv = wb[pl.ds(i * 16, 16)]
      for t in range(16):
        e = i * 16 + t
        rows[e] = rows[e] * jnp.broadcast_to(wv[t], (HALF,))

  def chunk(m, _):
    r0 = base_rows + m * (2 * SUB)
    d1 = pltpu.async_copy(src2.at[pl.ds(r0, 2 * SUB)], idxb, lsem)
    d2 = pltpu.async_copy(dst2.at[pl.ds(r0, 2 * SUB)], dstb, lsem)
    d3 = pltpu.async_copy(w_hbm.at[pl.ds(r0 * 128, PR)], wb, lsem)
    d1.wait(); d2.wait(); d3.wait()

    for j in range(2 * SUB):
      for l in range(8):
        v = idxb[j, pl.ds(l * 16, 16)]
        idxb[j, pl.ds(l * 16, 16)] = v * 2 + c

    def gathers(h):
      return [
          pltpu.async_copy(hview.at[idxb.at[j]],
                           rows.at[pl.ds(j * 128, 128)], gsem)
          for j in range(h * SUB, (h + 1) * SUB)
      ]

    def scatters(h):
      return [
          pltpu.async_copy(rows.at[pl.ds(j * 128, 128)],
                           acc.at[dstb.at[j]], ssem, add=True)
          for j in range(h * SUB, (h + 1) * SUB)
      ]

    g0 = gathers(0)
    for d in g0:
      d.wait()
    g1 = gathers(1)
    scale_half(0)
    for d in g1:
      d.wait()
    s0 = scatters(0)
    scale_half(1)
    for d in s0:
      d.wait()
    s1 = scatters(1)
    for d in s1:
      d.wait()
    return 0

  lax.fori_loop(0, NITER, chunk, 0)
  plsc.subcore_barrier()

  @pl.when(s == 0)
  def _():
    pltpu.sync_copy(acc, out.at[:, c])


def _sc_edge(hview, src2, dst2, wp, zeros2):
  fn = pl.kernel(
      _edge_body,
      out_type=jax.ShapeDtypeStruct((NPD, NC, HALF), jnp.float32),
      mesh=_mesh(),
      scratch_types=[
          pltpu.VMEM((2 * SUB, 128), jnp.int32),
          pltpu.VMEM((2 * SUB, 128), jnp.int32),
          pltpu.VMEM((PR,), jnp.float32),
          pltpu.VMEM((PR, HALF), jnp.float32),
          pltpu.SemaphoreType.DMA,
          pltpu.SemaphoreType.DMA,
          pltpu.SemaphoreType.DMA,
          pltpu.VMEM_SHARED((NPD, HALF), jnp.float32),
      ],
      compiler_params=pltpu.CompilerParams(needs_layout_passes=False,
                                           use_tc_tiling_on_sc=False),
  )
  return fn(hview, src2, dst2, wp, zeros2)


def _dis_body(degp_ref, dis_ref):
  deg = 1.0 + jnp.sum(degp_ref[...], axis=0)
  dis_ref[...] = lax.rsqrt(deg)


def _tc_dis(degp3):
  nblk = NPD // NB
  return pl.pallas_call(
      _dis_body,
      grid=(nblk,),
      in_specs=[pl.BlockSpec((NW, BKB, 128), lambda i: (0, i, 0))],
      out_specs=pl.BlockSpec((BKB, 128), lambda i: (i, 0)),
      out_shape=jax.ShapeDtypeStruct((NPD // 128, 128), jnp.float32),
  )(degp3)


def _pre_body(x_ref, dis_ref, w0_ref, b0_ref, w1a_ref, w1b_ref, b1_ref,
              g0w_ref, hpk_ref):
  x = x_ref[...]
  h0 = jax.nn.relu(jnp.dot(x, w0_ref[...], preferred_element_type=jnp.float32)
                   + b0_ref[...])
  h = jax.nn.relu(jnp.dot(h0, w1a_ref[...], preferred_element_type=jnp.float32)
                  + jnp.dot(x, w1b_ref[...], preferred_element_type=jnp.float32)
                  + b1_ref[...])
  for q in range(4):
    hq = h[q * QR:(q + 1) * QR]
    dq = dis_ref[:, q * D:q * D + 1]
    htq = dq * jnp.dot(hq, g0w_ref[...], preferred_element_type=jnp.float32)
    hpk_ref[:, q * D:(q + 1) * D] = htq


def _tc_pre(xshuf, dis32pk, W0, b0, W1a, W1b, b1, G0W):
  nblk = NPD // NB
  return pl.pallas_call(
      _pre_body,
      grid=(nblk,),
      in_specs=[
          pl.BlockSpec((NB, F_IN), lambda i: (i, 0)),
          pl.BlockSpec((QR, 128), lambda i: (i, 0)),
          pl.BlockSpec((F_IN, D), lambda i: (0, 0)),
          pl.BlockSpec((1, D), lambda i: (0, 0)),
          pl.BlockSpec((D, D), lambda i: (0, 0)),
          pl.BlockSpec((F_IN, D), lambda i: (0, 0)),
          pl.BlockSpec((1, D), lambda i: (0, 0)),
          pl.BlockSpec((D, D), lambda i: (0, 0)),
      ],
      out_specs=pl.BlockSpec((QR, 128), lambda i: (i, 0)),
      out_shape=jax.ShapeDtypeStruct((PKN, 128), jnp.float32),
  )(xshuf, dis32pk, W0, b0, W1a, W1b, b1, G0W)


def _layer_body(s_ref, h_ref, dis_ref, b4_ref, wblk_ref, o_ref):
  dis = dis_ref[...]
  h = jax.nn.relu(dis * (s_ref[...] + h_ref[...]) + b4_ref[...])
  o_ref[...] = dis * jnp.dot(h, wblk_ref[...],
                             preferred_element_type=jnp.float32)


def _tc_layer(spk, hpk, dis32pk, b4, Wblk):
  nblk = NPD // NB
  return pl.pallas_call(
      _layer_body,
      grid=(nblk,),
      in_specs=[
          pl.BlockSpec((QR, 128), lambda i: (i, 0)),
          pl.BlockSpec((QR, 128), lambda i: (i, 0)),
          pl.BlockSpec((QR, 128), lambda i: (i, 0)),
          pl.BlockSpec((1, 128), lambda i: (0, 0)),
          pl.BlockSpec((128, 128), lambda i: (0, 0)),
      ],
      out_specs=pl.BlockSpec((QR, 128), lambda i: (i, 0)),
      out_shape=jax.ShapeDtypeStruct((PKN, 128), jnp.float32),
  )(spk, hpk, dis32pk, b4, Wblk)


def _pool_body(s_ref, h_ref, dis_ref, b4_ref, batch_ref, sums_ref, cnt_ref):
  dis = dis_ref[...]
  h3 = jax.nn.relu(dis * (s_ref[...] + h_ref[...]) + b4_ref[...])
  ids = lax.broadcasted_iota(jnp.int32, (1, G), 1)
  ones = jnp.ones((QR, 1), jnp.float32)
  part = jnp.zeros((G, D), jnp.float32)
  pcnt = jnp.zeros((G, 1), jnp.float32)
  for q in range(4):
    h3q = h3[:, q * D:(q + 1) * D]
    bq = batch_ref[:, q * D:q * D + 1]
    mask = (bq == ids).astype(jnp.float32)
    part = part + lax.dot_general(mask, h3q, (((0,), (0,)), ((), ())),
                                  preferred_element_type=jnp.float32)
    pcnt = pcnt + lax.dot_general(mask, ones, (((0,), (0,)), ((), ())),
                                  preferred_element_type=jnp.float32)

  @pl.when(pl.program_id(0) == 0)
  def _():
    sums_ref[...] = jnp.zeros_like(sums_ref)
    cnt_ref[...] = jnp.zeros_like(cnt_ref)

  sums_ref[...] += part
  cnt_ref[...] += pcnt


def _tc_pool(spk, hpk, dis32pk, b4, batch32pk):
  nblk = NPD // NB
  return pl.pallas_call(
      _pool_body,
      grid=(nblk,),
      in_specs=[
          pl.BlockSpec((QR, 128), lambda i: (i, 0)),
          pl.BlockSpec((QR, 128), lambda i: (i, 0)),
          pl.BlockSpec((QR, 128), lambda i: (i, 0)),
          pl.BlockSpec((1, 128), lambda i: (0, 0)),
          pl.BlockSpec((QR, 128), lambda i: (i, 0)),
      ],
      out_specs=[
          pl.BlockSpec((G, D), lambda i: (0, 0)),
          pl.BlockSpec((G, 1), lambda i: (0, 0)),
      ],
      out_shape=[
          jax.ShapeDtypeStruct((G, D), jnp.float32),
          jax.ShapeDtypeStruct((G, 1), jnp.float32),
      ],
  )(spk, hpk, dis32pk, b4, batch32pk)


def _head_body(sums_ref, cnt_ref, p0w_ref, p0b_ref, p1a_ref, p1b_ref,
               p1bias_ref, out_ref):
  g = sums_ref[...] / jnp.maximum(cnt_ref[...], 1.0)
  p = jnp.dot(g, p0w_ref[...], preferred_element_type=jnp.float32) + p0b_ref[...]
  z = (jnp.dot(jax.nn.relu(p), p1a_ref[...], preferred_element_type=jnp.float32)
       + jnp.dot(p, p1b_ref[...], preferred_element_type=jnp.float32)
       + p1bias_ref[...])
  out_ref[...] = 1.0 / (1.0 + jnp.exp(-z))


def _tc_head(sums, cnt, P0_W, p0b, P1a, P1b, p1bias):
  return pl.pallas_call(
      _head_body,
      out_shape=jax.ShapeDtypeStruct((G, 1), jnp.float32),
  )(sums, cnt, P0_W, p0b, P1a, P1b, p1bias)


def kernel(x, edge_indices, edge_weights, batch, MLP0_W, MLP0_b, MLP1_W,
           MLP1_b, G0_W, G0_b, G1_W, G1_b, G2_W, G2_b, P0_W, P0_b, P1_W, P1_b):
  src = edge_indices[0]
  dst = edge_indices[1]
  pad = EPAD - E
  srcp = jnp.concatenate([src, jnp.zeros((pad,), jnp.int32)])
  dstp = jnp.concatenate([dst, jnp.zeros((pad,), jnp.int32)])
  wp = jnp.concatenate([edge_weights, jnp.zeros((pad,), jnp.float32)])
  src2 = srcp.reshape(-1, 128)
  dst2 = dstp.reshape(-1, 128)
  zeros1 = jnp.zeros((NPD,), jnp.float32)
  zeros2 = jnp.zeros((NPD, HALF), jnp.float32)
  xp = jnp.concatenate([x, jnp.zeros((NPD - N, F_IN), jnp.float32)])
  xshuf = xp.reshape(NPD // NB, QR, 4, F_IN).transpose(0, 2, 1, 3)
  xshuf = xshuf.reshape(NPD, F_IN)
  batchp = jnp.concatenate([batch, jnp.full((NPD - N,), G, jnp.int32)])
  batch32pk = jnp.broadcast_to(batchp[:, None], (NPD, D)).reshape(PKN, 128)

  b0 = MLP0_b[None, :]
  b1 = MLP1_b[None, :]
  W1a = MLP1_W[:D]
  W1b = MLP1_W[D:]
  eye4 = jnp.eye(4, dtype=jnp.float32)
  wblk = (jnp.kron(eye4, G1_W), jnp.kron(eye4, G2_W))
  b4 = (jnp.tile(G0_b[None, :], (1, 4)), jnp.tile(G1_b[None, :], (1, 4)),
        jnp.tile(G2_b[None, :], (1, 4)))
  p0b = P0_b[None, :]
  P1a = P1_W[:D]
  P1b = P1_W[D:]
  p1bias = P1_b[None, :]

  degp = _sc_degree(dstp, wp, zeros1)
  degp3 = degp.reshape(NW, NPD // 128, 128)
  disv = _tc_dis(degp3)
  disflat = disv.reshape(NPD)
  dis32pk = jnp.broadcast_to(disflat[:, None], (NPD, D)).reshape(PKN, 128)

  hpk = _tc_pre(xshuf, dis32pk, MLP0_W, b0, W1a, W1b, b1, G0_W)

  def edge(hpk):
    hview = hpk.reshape(NC * NPD, HALF)
    sout = _sc_edge(hview, src2, dst2, wp, zeros2)
    return sout.reshape(PKN, 128)

  spk = edge(hpk)
  hpk = _tc_layer(spk, hpk, dis32pk, b4[0], wblk[0])
  spk = edge(hpk)
  hpk = _tc_layer(spk, hpk, dis32pk, b4[1], wblk[1])
  spk = edge(hpk)
  sums, cnt = _tc_pool(spk, hpk, dis32pk, b4[2], batch32pk)

  return _tc_head(sums, cnt, P0_W, p0b, P1a, P1b, p1bias)


# speedup vs baseline: 1.0557x; 1.0557x over previous
import jax
import jax.numpy as jnp
from jax import lax
from jax.experimental import pallas as pl
from jax.experimental.pallas import tpu as pltpu
from jax.experimental.pallas import tpu_sc as plsc

N = 100000
E = 1600000
F_IN = 128
D = 32
G = 256

NC = 2
NS = 16
NW = NC * NS

CH = 1024
SUB = CH // 128
PR = CH
EPAD = 1605632
EPS = EPAD // NS
NITER = EPS // CH
EPW = EPAD // NW
DCH = 7168
NPD = 100352

HALF = 16
NB = 2048
QR = NB // 4
PKN = NPD * D // 128
BKB = NB // 128


def _mesh():
  return plsc.VectorSubcoreMesh(core_axis_name="c", subcore_axis_name="s",
                                num_cores=NC, num_subcores=NS)


def _deg_body(dst_hbm, w_hbm, zero_hbm, out_hbm, acc, dstb, wb, sem):
  c = lax.axis_index("c")
  s = lax.axis_index("s")
  wid = s * NC + c
  pltpu.sync_copy(zero_hbm, acc)
  base = wid * EPW

  def chunk(k, _):
    off = base + k * DCH
    pltpu.async_copy(dst_hbm.at[pl.ds(off, DCH)], dstb, sem).wait()
    pltpu.async_copy(w_hbm.at[pl.ds(off, DCH)], wb, sem).wait()

    def inner(i, _):
      idx = dstb[pl.ds(i * 16, 16)]
      val = wb[pl.ds(i * 16, 16)]
      plsc.addupdate_scatter(acc, [idx], val)
      return 0

    lax.fori_loop(0, DCH // 16, inner, 0, unroll=8)
    return 0

  lax.fori_loop(0, EPW // DCH, chunk, 0)
  pltpu.sync_copy(acc, out_hbm.at[wid])


def _sc_degree(dstp, wp, zeros1):
  fn = pl.kernel(
      _deg_body,
      out_type=jax.ShapeDtypeStruct((NW, NPD), jnp.float32),
      mesh=_mesh(),
      scratch_types=[
          pltpu.VMEM((NPD,), jnp.float32),
          pltpu.VMEM((DCH,), jnp.int32),
          pltpu.VMEM((DCH,), jnp.float32),
          pltpu.SemaphoreType.DMA,
      ],
      compiler_params=pltpu.CompilerParams(needs_layout_passes=False),
  )
  return fn(dstp, wp, zeros1)


def _edge_body(hview, src2, dst2, w_hbm, zero_hbm, out,
               idxb, dstb, wb, rows, lsem, gsem, ssem, acc):
  c = lax.axis_index("c")
  s = lax.axis_index("s")

  for k in range(NPD // CH):
    @pl.when(k % NS == s)
    def _():
      pltpu.sync_copy(zero_hbm.at[pl.ds(k * CH, CH)], acc.at[pl.ds(k * CH, CH)])
  plsc.subcore_barrier()

  base_rows = s * (EPS // 128)

  def chunk(k, _):
    r0 = base_rows + k * SUB
    d1 = pltpu.async_copy(src2.at[pl.ds(r0, SUB)], idxb, lsem)
    d2 = pltpu.async_copy(dst2.at[pl.ds(r0, SUB)], dstb, lsem)
    d3 = pltpu.async_copy(w_hbm.at[pl.ds(r0 * 128, CH)], wb, lsem)
    d1.wait()

    for j in range(SUB):
      for l in range(8):
        v = idxb[j, pl.ds(l * 16, 16)]
        idxb[j, pl.ds(l * 16, 16)] = v * 2 + c

    descs = [
        pltpu.async_copy(hview.at[idxb.at[j]],
                         rows.at[pl.ds(j * 128, 128)], gsem)
        for j in range(SUB)
    ]
    d2.wait(); d3.wait()
    for d in descs:
      d.wait()

    @plsc.parallel_loop(0, CH // 16, 1, unroll=2)
    def _(i):
      wv = wb[pl.ds(i * 16, 16)]
      for t in range(16):
        e = i * 16 + t
        rows[e] = rows[e] * jnp.broadcast_to(wv[t], (HALF,))

    sdescs = [
        pltpu.async_copy(rows.at[pl.ds(j * 128, 128)],
                         acc.at[dstb.at[j]], ssem, add=True)
        for j in range(SUB)
    ]
    for d in sdescs:
      d.wait()
    return 0

  lax.fori_loop(0, NITER, chunk, 0)
  plsc.subcore_barrier()

  @pl.when(s == 0)
  def _():
    pltpu.sync_copy(acc, out.at[:, c])


def _sc_edge(hview, src2, dst2, wp, zeros2):
  fn = pl.kernel(
      _edge_body,
      out_type=jax.ShapeDtypeStruct((NPD, NC, HALF), jnp.float32),
      mesh=_mesh(),
      scratch_types=[
          pltpu.VMEM((SUB, 128), jnp.int32),
          pltpu.VMEM((SUB, 128), jnp.int32),
          pltpu.VMEM((CH,), jnp.float32),
          pltpu.VMEM((CH, HALF), jnp.float32),
          pltpu.SemaphoreType.DMA,
          pltpu.SemaphoreType.DMA,
          pltpu.SemaphoreType.DMA,
          pltpu.VMEM_SHARED((NPD, HALF), jnp.float32),
      ],
      compiler_params=pltpu.CompilerParams(needs_layout_passes=False,
                                           use_tc_tiling_on_sc=False),
  )
  return fn(hview, src2, dst2, wp, zeros2)


def _dis_body(degp_ref, dis_ref):
  deg = 1.0 + jnp.sum(degp_ref[...], axis=0)
  dis_ref[...] = lax.rsqrt(deg)


def _tc_dis(degp3):
  nblk = NPD // NB
  return pl.pallas_call(
      _dis_body,
      grid=(nblk,),
      in_specs=[pl.BlockSpec((NW, BKB, 128), lambda i: (0, i, 0))],
      out_specs=pl.BlockSpec((BKB, 128), lambda i: (i, 0)),
      out_shape=jax.ShapeDtypeStruct((NPD // 128, 128), jnp.float32),
  )(degp3)


def _pre_body(x_ref, dis_ref, w0_ref, b0_ref, w1a_ref, w1b_ref, b1_ref,
              g0w_ref, hpk_ref):
  x = x_ref[...]
  h0 = jax.nn.relu(jnp.dot(x, w0_ref[...], preferred_element_type=jnp.float32)
                   + b0_ref[...])
  h = jax.nn.relu(jnp.dot(h0, w1a_ref[...], preferred_element_type=jnp.float32)
                  + jnp.dot(x, w1b_ref[...], preferred_element_type=jnp.float32)
                  + b1_ref[...])
  for q in range(4):
    hq = h[q * QR:(q + 1) * QR]
    dq = dis_ref[:, q * D:q * D + 1]
    htq = dq * jnp.dot(hq, g0w_ref[...], preferred_element_type=jnp.float32)
    hpk_ref[:, q * D:(q + 1) * D] = htq


def _tc_pre(xshuf, dis32pk, W0, b0, W1a, W1b, b1, G0W):
  nblk = NPD // NB
  return pl.pallas_call(
      _pre_body,
      grid=(nblk,),
      in_specs=[
          pl.BlockSpec((NB, F_IN), lambda i: (i, 0)),
          pl.BlockSpec((QR, 128), lambda i: (i, 0)),
          pl.BlockSpec((F_IN, D), lambda i: (0, 0)),
          pl.BlockSpec((1, D), lambda i: (0, 0)),
          pl.BlockSpec((D, D), lambda i: (0, 0)),
          pl.BlockSpec((F_IN, D), lambda i: (0, 0)),
          pl.BlockSpec((1, D), lambda i: (0, 0)),
          pl.BlockSpec((D, D), lambda i: (0, 0)),
      ],
      out_specs=pl.BlockSpec((QR, 128), lambda i: (i, 0)),
      out_shape=jax.ShapeDtypeStruct((PKN, 128), jnp.float32),
  )(xshuf, dis32pk, W0, b0, W1a, W1b, b1, G0W)


def _layer_body(s_ref, h_ref, dis_ref, b4_ref, wblk_ref, o_ref):
  dis = dis_ref[...]
  h = jax.nn.relu(dis * (s_ref[...] + h_ref[...]) + b4_ref[...])
  o_ref[...] = dis * jnp.dot(h, wblk_ref[...],
                             preferred_element_type=jnp.float32)


def _tc_layer(spk, hpk, dis32pk, b4, Wblk):
  nblk = NPD // NB
  return pl.pallas_call(
      _layer_body,
      grid=(nblk,),
      in_specs=[
          pl.BlockSpec((QR, 128), lambda i: (i, 0)),
          pl.BlockSpec((QR, 128), lambda i: (i, 0)),
          pl.BlockSpec((QR, 128), lambda i: (i, 0)),
          pl.BlockSpec((1, 128), lambda i: (0, 0)),
          pl.BlockSpec((128, 128), lambda i: (0, 0)),
      ],
      out_specs=pl.BlockSpec((QR, 128), lambda i: (i, 0)),
      out_shape=jax.ShapeDtypeStruct((PKN, 128), jnp.float32),
  )(spk, hpk, dis32pk, b4, Wblk)


def _pool_body(s_ref, h_ref, dis_ref, b4_ref, batch_ref, sums_ref, cnt_ref):
  dis = dis_ref[...]
  h3 = jax.nn.relu(dis * (s_ref[...] + h_ref[...]) + b4_ref[...])
  ids = lax.broadcasted_iota(jnp.int32, (1, G), 1)
  ones = jnp.ones((QR, 1), jnp.float32)
  part = jnp.zeros((G, D), jnp.float32)
  pcnt = jnp.zeros((G, 1), jnp.float32)
  for q in range(4):
    h3q = h3[:, q * D:(q + 1) * D]
    bq = batch_ref[:, q * D:q * D + 1]
    mask = (bq == ids).astype(jnp.float32)
    part = part + lax.dot_general(mask, h3q, (((0,), (0,)), ((), ())),
                                  preferred_element_type=jnp.float32)
    pcnt = pcnt + lax.dot_general(mask, ones, (((0,), (0,)), ((), ())),
                                  preferred_element_type=jnp.float32)

  @pl.when(pl.program_id(0) == 0)
  def _():
    sums_ref[...] = jnp.zeros_like(sums_ref)
    cnt_ref[...] = jnp.zeros_like(cnt_ref)

  sums_ref[...] += part
  cnt_ref[...] += pcnt


def _tc_pool(spk, hpk, dis32pk, b4, batch32pk):
  nblk = NPD // NB
  return pl.pallas_call(
      _pool_body,
      grid=(nblk,),
      in_specs=[
          pl.BlockSpec((QR, 128), lambda i: (i, 0)),
          pl.BlockSpec((QR, 128), lambda i: (i, 0)),
          pl.BlockSpec((QR, 128), lambda i: (i, 0)),
          pl.BlockSpec((1, 128), lambda i: (0, 0)),
          pl.BlockSpec((QR, 128), lambda i: (i, 0)),
      ],
      out_specs=[
          pl.BlockSpec((G, D), lambda i: (0, 0)),
          pl.BlockSpec((G, 1), lambda i: (0, 0)),
      ],
      out_shape=[
          jax.ShapeDtypeStruct((G, D), jnp.float32),
          jax.ShapeDtypeStruct((G, 1), jnp.float32),
      ],
  )(spk, hpk, dis32pk, b4, batch32pk)


def _head_body(sums_ref, cnt_ref, p0w_ref, p0b_ref, p1a_ref, p1b_ref,
               p1bias_ref, out_ref):
  g = sums_ref[...] / jnp.maximum(cnt_ref[...], 1.0)
  p = jnp.dot(g, p0w_ref[...], preferred_element_type=jnp.float32) + p0b_ref[...]
  z = (jnp.dot(jax.nn.relu(p), p1a_ref[...], preferred_element_type=jnp.float32)
       + jnp.dot(p, p1b_ref[...], preferred_element_type=jnp.float32)
       + p1bias_ref[...])
  out_ref[...] = 1.0 / (1.0 + jnp.exp(-z))


def _tc_head(sums, cnt, P0_W, p0b, P1a, P1b, p1bias):
  return pl.pallas_call(
      _head_body,
      out_shape=jax.ShapeDtypeStruct((G, 1), jnp.float32),
  )(sums, cnt, P0_W, p0b, P1a, P1b, p1bias)


def kernel(x, edge_indices, edge_weights, batch, MLP0_W, MLP0_b, MLP1_W,
           MLP1_b, G0_W, G0_b, G1_W, G1_b, G2_W, G2_b, P0_W, P0_b, P1_W, P1_b):
  src = edge_indices[0]
  dst = edge_indices[1]
  pad = EPAD - E
  srcp = jnp.concatenate([src, jnp.zeros((pad,), jnp.int32)])
  dstp = jnp.concatenate([dst, jnp.zeros((pad,), jnp.int32)])
  wp = jnp.concatenate([edge_weights, jnp.zeros((pad,), jnp.float32)])
  src2 = srcp.reshape(-1, 128)
  dst2 = dstp.reshape(-1, 128)
  zeros1 = jnp.zeros((NPD,), jnp.float32)
  zeros2 = jnp.zeros((NPD, HALF), jnp.float32)
  xp = jnp.concatenate([x, jnp.zeros((NPD - N, F_IN), jnp.float32)])
  xshuf = xp.reshape(NPD // NB, QR, 4, F_IN).transpose(0, 2, 1, 3)
  xshuf = xshuf.reshape(NPD, F_IN)
  batchp = jnp.concatenate([batch, jnp.full((NPD - N,), G, jnp.int32)])
  batch32pk = jnp.broadcast_to(batchp[:, None], (NPD, D)).reshape(PKN, 128)

  b0 = MLP0_b[None, :]
  b1 = MLP1_b[None, :]
  W1a = MLP1_W[:D]
  W1b = MLP1_W[D:]
  eye4 = jnp.eye(4, dtype=jnp.float32)
  wblk = (jnp.kron(eye4, G1_W), jnp.kron(eye4, G2_W))
  b4 = (jnp.tile(G0_b[None, :], (1, 4)), jnp.tile(G1_b[None, :], (1, 4)),
        jnp.tile(G2_b[None, :], (1, 4)))
  p0b = P0_b[None, :]
  P1a = P1_W[:D]
  P1b = P1_W[D:]
  p1bias = P1_b[None, :]

  degp = _sc_degree(dstp, wp, zeros1)
  degp3 = degp.reshape(NW, NPD // 128, 128)
  disv = _tc_dis(degp3)
  disflat = disv.reshape(NPD)
  dis32pk = jnp.broadcast_to(disflat[:, None], (NPD, D)).reshape(PKN, 128)

  hpk = _tc_pre(xshuf, dis32pk, MLP0_W, b0, W1a, W1b, b1, G0_W)

  def edge(hpk):
    hview = hpk.reshape(NC * NPD, HALF)
    sout = _sc_edge(hview, src2, dst2, wp, zeros2)
    return sout.reshape(PKN, 128)

  spk = edge(hpk)
  hpk = _tc_layer(spk, hpk, dis32pk, b4[0], wblk[0])
  spk = edge(hpk)
  hpk = _tc_layer(spk, hpk, dis32pk, b4[1], wblk[1])
  spk = edge(hpk)
  sums, cnt = _tc_pool(spk, hpk, dis32pk, b4[2], batch32pk)

  return _tc_head(sums, cnt, P0_W, p0b, P1a, P1b, p1bias)


# speedup vs baseline: 1.1425x; 1.0821x over previous
import jax
import jax.numpy as jnp
from jax import lax
from jax.experimental import pallas as pl
from jax.experimental.pallas import tpu as pltpu
from jax.experimental.pallas import tpu_sc as plsc

N = 100000
E = 1600000
F_IN = 128
D = 32
G = 256

NC = 2
NS = 16
NW = NC * NS

CH = 1024
SUB = CH // 128
EPAD = 1605632
EPS = EPAD // NS
NITER = EPS // CH
EPW = EPAD // NW
DCH = 7168
NPD = 100352

HALF = 16
NB = 2048
QR = NB // 4
PKN = NPD * D // 128
BKB = NB // 128


def _mesh():
  return plsc.VectorSubcoreMesh(core_axis_name="c", subcore_axis_name="s",
                                num_cores=NC, num_subcores=NS)


def _deg_body(dst_hbm, w_hbm, zero_hbm, out_hbm, acc, dstb, wb, sem):
  c = lax.axis_index("c")
  s = lax.axis_index("s")
  wid = s * NC + c
  pltpu.sync_copy(zero_hbm, acc)
  base = wid * EPW

  def chunk(k, _):
    off = base + k * DCH
    pltpu.async_copy(dst_hbm.at[pl.ds(off, DCH)], dstb, sem).wait()
    pltpu.async_copy(w_hbm.at[pl.ds(off, DCH)], wb, sem).wait()

    def inner(i, _):
      idx = dstb[pl.ds(i * 16, 16)]
      val = wb[pl.ds(i * 16, 16)]
      plsc.addupdate_scatter(acc, [idx], val)
      return 0

    lax.fori_loop(0, DCH // 16, inner, 0, unroll=8)
    return 0

  lax.fori_loop(0, EPW // DCH, chunk, 0)
  pltpu.sync_copy(acc, out_hbm.at[wid])


def _sc_degree(dstp, wp, zeros1):
  fn = pl.kernel(
      _deg_body,
      out_type=jax.ShapeDtypeStruct((NW, NPD), jnp.float32),
      mesh=_mesh(),
      scratch_types=[
          pltpu.VMEM((NPD,), jnp.float32),
          pltpu.VMEM((DCH,), jnp.int32),
          pltpu.VMEM((DCH,), jnp.float32),
          pltpu.SemaphoreType.DMA,
      ],
      compiler_params=pltpu.CompilerParams(needs_layout_passes=False),
  )
  return fn(dstp, wp, zeros1)


def _edge_body(hview, src2, dst2, w_hbm, zero_hbm, out,
               idxb, dstb, wb, rows, lsem, gsem, ssem, acc):
  c = lax.axis_index("c")
  s = lax.axis_index("s")

  for k in range(NPD // CH):
    @pl.when(k % NS == s)
    def _():
      pltpu.sync_copy(zero_hbm.at[pl.ds(k * CH, CH)], acc.at[pl.ds(k * CH, CH)])
  plsc.subcore_barrier()

  base_rows = s * (EPS // 128)

  def drain_scatters():
    for j in range(SUB):
      pltpu.make_async_copy(rows.at[pl.ds(j * 128, 128)],
                            acc.at[dstb.at[j]], ssem).wait()

  def chunk(k, _):
    r0 = base_rows + k * SUB
    d1 = pltpu.async_copy(src2.at[pl.ds(r0, SUB)], idxb, lsem)
    d3 = pltpu.async_copy(w_hbm.at[pl.ds(r0 * 128, CH)], wb, lsem)

    @pl.when(k > 0)
    def _():
      drain_scatters()
    d2 = pltpu.async_copy(dst2.at[pl.ds(r0, SUB)], dstb, lsem)
    d1.wait()

    for j in range(SUB):
      for l in range(8):
        v = idxb[j, pl.ds(l * 16, 16)]
        idxb[j, pl.ds(l * 16, 16)] = v * 2 + c

    descs = [
        pltpu.async_copy(hview.at[idxb.at[j]],
                         rows.at[pl.ds(j * 128, 128)], gsem)
        for j in range(SUB)
    ]
    d2.wait(); d3.wait()
    for d in descs:
      d.wait()

    @plsc.parallel_loop(0, CH // 16, 1, unroll=2)
    def _(i):
      wv = wb[pl.ds(i * 16, 16)]
      for t in range(16):
        e = i * 16 + t
        rows[e] = rows[e] * jnp.broadcast_to(wv[t], (HALF,))

    for j in range(SUB):
      pltpu.async_copy(rows.at[pl.ds(j * 128, 128)],
                       acc.at[dstb.at[j]], ssem, add=True)
    return 0

  lax.fori_loop(0, NITER, chunk, 0)
  drain_scatters()
  plsc.subcore_barrier()

  @pl.when(s == 0)
  def _():
    pltpu.sync_copy(acc, out.at[:, c])


def _sc_edge(hview, src2, dst2, wp, zeros2):
  fn = pl.kernel(
      _edge_body,
      out_type=jax.ShapeDtypeStruct((NPD, NC, HALF), jnp.float32),
      mesh=_mesh(),
      scratch_types=[
          pltpu.VMEM((SUB, 128), jnp.int32),
          pltpu.VMEM((SUB, 128), jnp.int32),
          pltpu.VMEM((CH,), jnp.float32),
          pltpu.VMEM((CH, HALF), jnp.float32),
          pltpu.SemaphoreType.DMA,
          pltpu.SemaphoreType.DMA,
          pltpu.SemaphoreType.DMA,
          pltpu.VMEM_SHARED((NPD, HALF), jnp.float32),
      ],
      compiler_params=pltpu.CompilerParams(needs_layout_passes=False,
                                           use_tc_tiling_on_sc=False),
  )
  return fn(hview, src2, dst2, wp, zeros2)


def _dis_body(degp_ref, dis_ref):
  deg = 1.0 + jnp.sum(degp_ref[...], axis=0)
  dis_ref[...] = lax.rsqrt(deg)


def _tc_dis(degp3):
  nblk = NPD // NB
  return pl.pallas_call(
      _dis_body,
      grid=(nblk,),
      in_specs=[pl.BlockSpec((NW, BKB, 128), lambda i: (0, i, 0))],
      out_specs=pl.BlockSpec((BKB, 128), lambda i: (i, 0)),
      out_shape=jax.ShapeDtypeStruct((NPD // 128, 128), jnp.float32),
  )(degp3)


def _pre_body(x_ref, dis_ref, w0_ref, b0_ref, w1a_ref, w1b_ref, b1_ref,
              g0w_ref, hpk_ref):
  x = x_ref[...]
  h0 = jax.nn.relu(jnp.dot(x, w0_ref[...], preferred_element_type=jnp.float32)
                   + b0_ref[...])
  h = jax.nn.relu(jnp.dot(h0, w1a_ref[...], preferred_element_type=jnp.float32)
                  + jnp.dot(x, w1b_ref[...], preferred_element_type=jnp.float32)
                  + b1_ref[...])
  for q in range(4):
    hq = h[q * QR:(q + 1) * QR]
    dq = dis_ref[:, q * D:q * D + 1]
    htq = dq * jnp.dot(hq, g0w_ref[...], preferred_element_type=jnp.float32)
    hpk_ref[:, q * D:(q + 1) * D] = htq


def _tc_pre(xshuf, dis32pk, W0, b0, W1a, W1b, b1, G0W):
  nblk = NPD // NB
  return pl.pallas_call(
      _pre_body,
      grid=(nblk,),
      in_specs=[
          pl.BlockSpec((NB, F_IN), lambda i: (i, 0)),
          pl.BlockSpec((QR, 128), lambda i: (i, 0)),
          pl.BlockSpec((F_IN, D), lambda i: (0, 0)),
          pl.BlockSpec((1, D), lambda i: (0, 0)),
          pl.BlockSpec((D, D), lambda i: (0, 0)),
          pl.BlockSpec((F_IN, D), lambda i: (0, 0)),
          pl.BlockSpec((1, D), lambda i: (0, 0)),
          pl.BlockSpec((D, D), lambda i: (0, 0)),
      ],
      out_specs=pl.BlockSpec((QR, 128), lambda i: (i, 0)),
      out_shape=jax.ShapeDtypeStruct((PKN, 128), jnp.float32),
  )(xshuf, dis32pk, W0, b0, W1a, W1b, b1, G0W)


def _layer_body(s_ref, h_ref, dis_ref, b4_ref, wblk_ref, o_ref):
  dis = dis_ref[...]
  h = jax.nn.relu(dis * (s_ref[...] + h_ref[...]) + b4_ref[...])
  o_ref[...] = dis * jnp.dot(h, wblk_ref[...],
                             preferred_element_type=jnp.float32)


def _tc_layer(spk, hpk, dis32pk, b4, Wblk):
  nblk = NPD // NB
  return pl.pallas_call(
      _layer_body,
      grid=(nblk,),
      in_specs=[
          pl.BlockSpec((QR, 128), lambda i: (i, 0)),
          pl.BlockSpec((QR, 128), lambda i: (i, 0)),
          pl.BlockSpec((QR, 128), lambda i: (i, 0)),
          pl.BlockSpec((1, 128), lambda i: (0, 0)),
          pl.BlockSpec((128, 128), lambda i: (0, 0)),
      ],
      out_specs=pl.BlockSpec((QR, 128), lambda i: (i, 0)),
      out_shape=jax.ShapeDtypeStruct((PKN, 128), jnp.float32),
  )(spk, hpk, dis32pk, b4, Wblk)


def _pool_body(s_ref, h_ref, dis_ref, b4_ref, batch_ref, sums_ref, cnt_ref):
  dis = dis_ref[...]
  h3 = jax.nn.relu(dis * (s_ref[...] + h_ref[...]) + b4_ref[...])
  ids = lax.broadcasted_iota(jnp.int32, (1, G), 1)
  ones = jnp.ones((QR, 1), jnp.float32)
  part = jnp.zeros((G, D), jnp.float32)
  pcnt = jnp.zeros((G, 1), jnp.float32)
  for q in range(4):
    h3q = h3[:, q * D:(q + 1) * D]
    bq = batch_ref[:, q * D:q * D + 1]
    mask = (bq == ids).astype(jnp.float32)
    part = part + lax.dot_general(mask, h3q, (((0,), (0,)), ((), ())),
                                  preferred_element_type=jnp.float32)
    pcnt = pcnt + lax.dot_general(mask, ones, (((0,), (0,)), ((), ())),
                                  preferred_element_type=jnp.float32)

  @pl.when(pl.program_id(0) == 0)
  def _():
    sums_ref[...] = jnp.zeros_like(sums_ref)
    cnt_ref[...] = jnp.zeros_like(cnt_ref)

  sums_ref[...] += part
  cnt_ref[...] += pcnt


def _tc_pool(spk, hpk, dis32pk, b4, batch32pk):
  nblk = NPD // NB
  return pl.pallas_call(
      _pool_body,
      grid=(nblk,),
      in_specs=[
          pl.BlockSpec((QR, 128), lambda i: (i, 0)),
          pl.BlockSpec((QR, 128), lambda i: (i, 0)),
          pl.BlockSpec((QR, 128), lambda i: (i, 0)),
          pl.BlockSpec((1, 128), lambda i: (0, 0)),
          pl.BlockSpec((QR, 128), lambda i: (i, 0)),
      ],
      out_specs=[
          pl.BlockSpec((G, D), lambda i: (0, 0)),
          pl.BlockSpec((G, 1), lambda i: (0, 0)),
      ],
      out_shape=[
          jax.ShapeDtypeStruct((G, D), jnp.float32),
          jax.ShapeDtypeStruct((G, 1), jnp.float32),
      ],
  )(spk, hpk, dis32pk, b4, batch32pk)


def _head_body(sums_ref, cnt_ref, p0w_ref, p0b_ref, p1a_ref, p1b_ref,
               p1bias_ref, out_ref):
  g = sums_ref[...] / jnp.maximum(cnt_ref[...], 1.0)
  p = jnp.dot(g, p0w_ref[...], preferred_element_type=jnp.float32) + p0b_ref[...]
  z = (jnp.dot(jax.nn.relu(p), p1a_ref[...], preferred_element_type=jnp.float32)
       + jnp.dot(p, p1b_ref[...], preferred_element_type=jnp.float32)
       + p1bias_ref[...])
  out_ref[...] = 1.0 / (1.0 + jnp.exp(-z))


def _tc_head(sums, cnt, P0_W, p0b, P1a, P1b, p1bias):
  return pl.pallas_call(
      _head_body,
      out_shape=jax.ShapeDtypeStruct((G, 1), jnp.float32),
  )(sums, cnt, P0_W, p0b, P1a, P1b, p1bias)


def kernel(x, edge_indices, edge_weights, batch, MLP0_W, MLP0_b, MLP1_W,
           MLP1_b, G0_W, G0_b, G1_W, G1_b, G2_W, G2_b, P0_W, P0_b, P1_W, P1_b):
  src = edge_indices[0]
  dst = edge_indices[1]
  pad = EPAD - E
  srcp = jnp.concatenate([src, jnp.zeros((pad,), jnp.int32)])
  dstp = jnp.concatenate([dst, jnp.zeros((pad,), jnp.int32)])
  wp = jnp.concatenate([edge_weights, jnp.zeros((pad,), jnp.float32)])
  src2 = srcp.reshape(-1, 128)
  dst2 = dstp.reshape(-1, 128)
  zeros1 = jnp.zeros((NPD,), jnp.float32)
  zeros2 = jnp.zeros((NPD, HALF), jnp.float32)
  xp = jnp.concatenate([x, jnp.zeros((NPD - N, F_IN), jnp.float32)])
  xshuf = xp.reshape(NPD // NB, QR, 4, F_IN).transpose(0, 2, 1, 3)
  xshuf = xshuf.reshape(NPD, F_IN)
  batchp = jnp.concatenate([batch, jnp.full((NPD - N,), G, jnp.int32)])
  batch32pk = jnp.broadcast_to(batchp[:, None], (NPD, D)).reshape(PKN, 128)

  b0 = MLP0_b[None, :]
  b1 = MLP1_b[None, :]
  W1a = MLP1_W[:D]
  W1b = MLP1_W[D:]
  eye4 = jnp.eye(4, dtype=jnp.float32)
  wblk = (jnp.kron(eye4, G1_W), jnp.kron(eye4, G2_W))
  b4 = (jnp.tile(G0_b[None, :], (1, 4)), jnp.tile(G1_b[None, :], (1, 4)),
        jnp.tile(G2_b[None, :], (1, 4)))
  p0b = P0_b[None, :]
  P1a = P1_W[:D]
  P1b = P1_W[D:]
  p1bias = P1_b[None, :]

  degp = _sc_degree(dstp, wp, zeros1)
  degp3 = degp.reshape(NW, NPD // 128, 128)
  disv = _tc_dis(degp3)
  disflat = disv.reshape(NPD)
  dis32pk = jnp.broadcast_to(disflat[:, None], (NPD, D)).reshape(PKN, 128)

  hpk = _tc_pre(xshuf, dis32pk, MLP0_W, b0, W1a, W1b, b1, G0_W)

  def edge(hpk):
    hview = hpk.reshape(NC * NPD, HALF)
    sout = _sc_edge(hview, src2, dst2, wp, zeros2)
    return sout.reshape(PKN, 128)

  spk = edge(hpk)
  hpk = _tc_layer(spk, hpk, dis32pk, b4[0], wblk[0])
  spk = edge(hpk)
  hpk = _tc_layer(spk, hpk, dis32pk, b4[1], wblk[1])
  spk = edge(hpk)
  sums, cnt = _tc_pool(spk, hpk, dis32pk, b4[2], batch32pk)

  return _tc_head(sums, cnt, P0_W, p0b, P1a, P1b, p1bias)
